# Initial kernel scaffold; baseline (speedup 1.0000x reference)
import jax
import jax.numpy as jnp
from jax.experimental import pallas as pl


def kernel(nv, ns, x, edge_index, ms1_w, ms1_b, ms2_w, ms2_b, mv_w, mv_b, us1_w, us1_b, us2_w, us2_b):
    raise NotImplementedError("write your pallas kernel here")


# speedup vs baseline: 2.6357x; 2.6357x over previous
import functools
import jax
import jax.numpy as jnp
from jax import lax
from jax.experimental import pallas as pl
from jax.experimental.pallas import tpu as pltpu

_EPS = 1e-05
_RC = 5.0
_L = 6
_LOG2 = 0.6931471805599453
_PI = 3.141592653589793
_D = 128
_NB = 1000
_EB = 2000


def _ssp(x):
    return jnp.maximum(x, 0.0) + jnp.log1p(jnp.exp(-jnp.abs(x))) - _LOG2


def _node1_body(ns_ref, w1_ref, b1_ref, w2_ref, b2_ref, pv_ref, ps_ref, pr_ref):
    h = _ssp(jnp.dot(ns_ref[...], w1_ref[...],
                     preferred_element_type=jnp.float32) + b1_ref[...])
    phi = jnp.dot(h, w2_ref[...], preferred_element_type=jnp.float32) + b2_ref[...]
    pv_ref[...] = phi[:, :_D]
    ps_ref[...] = phi[:, _D:2 * _D]
    pr_ref[...] = phi[:, 2 * _D:]


def _node1(ns, w1, b1, w2, b2):
    n = ns.shape[0]
    grid = n // _NB
    out = jax.ShapeDtypeStruct((n, _D), jnp.float32)
    return pl.pallas_call(
        _node1_body,
        grid=(grid,),
        in_specs=[
            pl.BlockSpec((_NB, _D), lambda i: (i, 0)),
            pl.BlockSpec((_D, _D), lambda i: (0, 0)),
            pl.BlockSpec((_D,), lambda i: (0,)),
            pl.BlockSpec((_D, 3 * _D), lambda i: (0, 0)),
            pl.BlockSpec((3 * _D,), lambda i: (0,)),
        ],
        out_specs=[pl.BlockSpec((_NB, _D), lambda i: (i, 0))] * 3,
        out_shape=[out, out, out],
    )(ns, w1, b1, w2, b2)


def _edgew_body(vec_ref, mw_ref, mb_ref, wv_ref, ws_ref, wr_ref, up_ref):
    vecp = vec_ref[...]
    vec = vecp[:, :3]
    r2 = jnp.sum(vec * vec, axis=1, keepdims=True) + _EPS
    r = jnp.sqrt(r2)
    n = lax.broadcasted_iota(jnp.int32, (1, _L), 1).astype(jnp.float32) + 1.0
    rbf = jnp.sqrt(2.0 / _RC) * jnp.sin(n * (_PI / _RC) * r) / r
    z = jnp.dot(rbf, mw_ref[...], preferred_element_type=jnp.float32) + mb_ref[...]
    w = 0.5 * (jnp.cos((_PI / _RC) * z) + 1.0) * (z < _RC).astype(jnp.float32)
    wv_ref[...] = w[:, :_D]
    ws_ref[...] = w[:, _D:2 * _D]
    wr_ref[...] = w[:, 2 * _D:]
    up_ref[...] = vecp / r


def _edgew(vecp, mw, mb):
    e = vecp.shape[0]
    grid = e // _EB
    out = jax.ShapeDtypeStruct((e, _D), jnp.float32)
    return pl.pallas_call(
        _edgew_body,
        grid=(grid,),
        in_specs=[
            pl.BlockSpec((_EB, 16), lambda i: (i, 0)),
            pl.BlockSpec((_L, 3 * _D), lambda i: (0, 0)),
            pl.BlockSpec((3 * _D,), lambda i: (0,)),
        ],
        out_specs=[pl.BlockSpec((_EB, _D), lambda i: (i, 0))] * 3
        + [pl.BlockSpec((_EB, 16), lambda i: (i, 0))],
        out_shape=[out, out, out, jax.ShapeDtypeStruct((e, 16), jnp.float32)],
    )(vecp, mw, mb)


def _node2_body(nvt_ref, dv0_ref, dv1_ref, dv2_ref, dsp_ref, ns_ref,
                u1_ref, ub1_ref, u2_ref, ub2_ref,
                vn_ref, sn_ref, gv_ref, gs_ref, ga_ref):
    vn_ref[0] = nvt_ref[0] + dv0_ref[0] + dv0_ref[1]
    vn_ref[1] = nvt_ref[1] + dv1_ref[0] + dv1_ref[1]
    vn_ref[2] = nvt_ref[2] + dv2_ref[0] + dv2_ref[1]
    s_new = ns_ref[...] + dsp_ref[0] + dsp_ref[1]
    sn_ref[...] = s_new
    h = _ssp(jnp.dot(s_new, u1_ref[...],
                     preferred_element_type=jnp.float32) + ub1_ref[...])
    g = jnp.dot(h, u2_ref[...], preferred_element_type=jnp.float32) + ub2_ref[...]
    gv_ref[...] = g[:, :_D]
    gs_ref[...] = g[:, _D:2 * _D]
    ga_ref[...] = g[:, 2 * _D:]


def _node2(nvt, dv0p, dv1p, dv2p, dsp, ns, u1, ub1, u2, ub2):
    n = ns.shape[0]
    grid = n // _NB
    out = jax.ShapeDtypeStruct((n, _D), jnp.float32)
    return pl.pallas_call(
        _node2_body,
        grid=(grid,),
        in_specs=[
            pl.BlockSpec((3, _NB, _D), lambda i: (0, i, 0)),
            pl.BlockSpec((2, _NB, _D), lambda i: (0, i, 0)),
            pl.BlockSpec((2, _NB, _D), lambda i: (0, i, 0)),
            pl.BlockSpec((2, _NB, _D), lambda i: (0, i, 0)),
            pl.BlockSpec((2, _NB, _D), lambda i: (0, i, 0)),
            pl.BlockSpec((_NB, _D), lambda i: (i, 0)),
            pl.BlockSpec((_D, _D), lambda i: (0, 0)),
            pl.BlockSpec((_D,), lambda i: (0,)),
            pl.BlockSpec((_D, 3 * _D), lambda i: (0, 0)),
            pl.BlockSpec((3 * _D,), lambda i: (0,)),
        ],
        out_specs=[pl.BlockSpec((3, _NB, _D), lambda i: (0, i, 0)),
                   pl.BlockSpec((_NB, _D), lambda i: (i, 0)),
                   pl.BlockSpec((_NB, _D), lambda i: (i, 0)),
                   pl.BlockSpec((_NB, _D), lambda i: (i, 0)),
                   pl.BlockSpec((_NB, _D), lambda i: (i, 0))],
        out_shape=[jax.ShapeDtypeStruct((3, n, _D), jnp.float32),
                   out, out, out, out],
    )(nvt, dv0p, dv1p, dv2p, dsp, ns, u1, ub1, u2, ub2)


def _final_body(vn_ref, sn_ref, uv0_ref, uv1_ref, uv2_ref,
                smv_ref, sms_ref, sma_ref, deg_ref, vo_ref, so_ref):
    deg = jnp.maximum(deg_ref[0, :, 0:1] + deg_ref[1, :, 0:1], 1.0)
    inv = 1.0 / deg
    uv0 = (uv0_ref[0] + uv0_ref[1]) * inv
    uv1 = (uv1_ref[0] + uv1_ref[1]) * inv
    uv2 = (uv2_ref[0] + uv2_ref[1]) * inv
    avv = (smv_ref[0] + smv_ref[1]) * inv
    asv = (sms_ref[0] + sms_ref[1]) * inv
    ass = (sma_ref[0] + sma_ref[1]) * inv
    vo_ref[0] = vn_ref[0] + uv0 * avv
    vo_ref[1] = vn_ref[1] + uv1 * avv
    vo_ref[2] = vn_ref[2] + uv2 * avv
    s2 = uv0 * uv0 + uv1 * uv1 + uv2 * uv2
    so_ref[...] = sn_ref[...] + s2 / (s2 + _EPS) * asv + ass


def _final(vnt, sn, uv0p, uv1p, uv2p, smvp, smsp, smap, degp):
    n = sn.shape[0]
    grid = n // _NB
    p2 = pl.BlockSpec((2, _NB, _D), lambda i: (0, i, 0))
    return pl.pallas_call(
        _final_body,
        grid=(grid,),
        in_specs=[
            pl.BlockSpec((3, _NB, _D), lambda i: (0, i, 0)),
            pl.BlockSpec((_NB, _D), lambda i: (i, 0)),
            p2, p2, p2, p2, p2, p2,
            pl.BlockSpec((2, _NB, 8), lambda i: (0, i, 0)),
        ],
        out_specs=[pl.BlockSpec((3, _NB, _D), lambda i: (0, i, 0)),
                   pl.BlockSpec((_NB, _D), lambda i: (i, 0))],
        out_shape=[jax.ShapeDtypeStruct((3, n, _D), jnp.float32),
                   jax.ShapeDtypeStruct((n, _D), jnp.float32)],
    )(vnt, sn, uv0p, uv1p, uv2p, smvp, smsp, smap, degp)


def _seg(vals, dst, n):
    full = jax.ops.segment_sum(vals, dst, num_segments=n)
    return jnp.stack([full, jnp.zeros_like(full)])


def kernel(nv, ns, x, edge_index, ms1_w, ms1_b, ms2_w, ms2_b, mv_w, mv_b,
           us1_w, us1_b, us2_w, us2_b):
    n = nv.shape[0]
    src = edge_index[0]
    dst = edge_index[1]
    nvt = jnp.transpose(nv, (2, 0, 1))
    x_pad = jnp.pad(x, ((0, 0), (0, 13)))

    phi_v, phi_s, phi_r = _node1(ns, ms1_w, ms1_b, ms2_w, ms2_b)

    vecp = x_pad[src] - x_pad[dst]
    w_v, w_s, w_r, u_pad = _edgew(vecp, mv_w, mv_b)

    pv = phi_v[src] * w_v
    pr = phi_r[src] * w_r
    dv0p = _seg(pv * nvt[0, src] + pr * u_pad[:, 0:1], dst, n)
    dv1p = _seg(pv * nvt[1, src] + pr * u_pad[:, 1:2], dst, n)
    dv2p = _seg(pv * nvt[2, src] + pr * u_pad[:, 2:3], dst, n)
    dsp = _seg(phi_s[src] * w_s, dst, n)

    vnt, s_new, g_v, g_s, g_a = _node2(nvt, dv0p, dv1p, dv2p, dsp, ns,
                                       us1_w, us1_b, us2_w, us2_b)

    uv0p = _seg(vnt[0, src], dst, n)
    uv1p = _seg(vnt[1, src], dst, n)
    uv2p = _seg(vnt[2, src], dst, n)
    smvp = _seg(g_v[src], dst, n)
    smsp = _seg(g_s[src], dst, n)
    smap = _seg(g_a[src], dst, n)
    degp = _seg(jnp.ones((src.shape[0], 8), jnp.float32), dst, n)

    vot, s_out = _final(vnt, s_new, uv0p, uv1p, uv2p, smvp, smsp, smap, degp)
    return jnp.transpose(vot, (1, 2, 0)), s_out


# speedup vs baseline: 12.4139x; 4.7099x over previous
import functools
import jax
import jax.numpy as jnp
from jax import lax
from jax.experimental import pallas as pl
from jax.experimental.pallas import tpu as pltpu
from jax.experimental.pallas import tpu_sc as plsc

_NC = 2
_NS = 16
_NW = _NC * _NS
_EPS = 1e-05
_RC = 5.0
_L = 6
_LOG2 = 0.6931471805599453
_PI = 3.141592653589793
_D = 128
_NB = 1000
_EB = 2048
_EP = 163840
_K = 128
_KD = 64
_SP = 8


def _ssp(x):
    return jnp.maximum(x, 0.0) + jnp.log1p(jnp.exp(-jnp.abs(x))) - _LOG2


def _node1_body(ns_ref, w1_ref, b1_ref, w2_ref, b2_ref, pv_ref, ps_ref, pr_ref):
    h = _ssp(jnp.dot(ns_ref[...], w1_ref[...],
                     preferred_element_type=jnp.float32) + b1_ref[...])
    phi = jnp.dot(h, w2_ref[...], preferred_element_type=jnp.float32) + b2_ref[...]
    pv_ref[...] = phi[:, :_D]
    ps_ref[...] = phi[:, _D:2 * _D]
    pr_ref[...] = phi[:, 2 * _D:]


def _node1(ns, w1, b1, w2, b2):
    n = ns.shape[0]
    grid = n // _NB
    out = jax.ShapeDtypeStruct((n, _D), jnp.float32)
    return pl.pallas_call(
        _node1_body,
        grid=(grid,),
        in_specs=[
            pl.BlockSpec((_NB, _D), lambda i: (i, 0)),
            pl.BlockSpec((_D, _D), lambda i: (0, 0)),
            pl.BlockSpec((_D,), lambda i: (0,)),
            pl.BlockSpec((_D, 3 * _D), lambda i: (0, 0)),
            pl.BlockSpec((3 * _D,), lambda i: (0,)),
        ],
        out_specs=[pl.BlockSpec((_NB, _D), lambda i: (i, 0))] * 3,
        out_shape=[out, out, out],
    )(ns, w1, b1, w2, b2)


def _edgew_body(e, vec_ref, mw_ref, mb_ref, wv_ref, ws_ref,
                wr0_ref, wr1_ref, wr2_ref):
    i = pl.program_id(0)
    vec = vec_ref[...][:, :3]
    r2 = jnp.sum(vec * vec, axis=1, keepdims=True) + _EPS
    r = jnp.sqrt(r2)
    u = vec / r
    nn = lax.broadcasted_iota(jnp.int32, (1, _L), 1).astype(jnp.float32) + 1.0
    rbf = jnp.sqrt(2.0 / _RC) * jnp.sin(nn * (_PI / _RC) * r) / r
    z = jnp.dot(rbf, mw_ref[...], preferred_element_type=jnp.float32) + mb_ref[...]
    w = 0.5 * (jnp.cos((_PI / _RC) * z) + 1.0) * (z < _RC).astype(jnp.float32)
    rowid = i * _EB + lax.broadcasted_iota(jnp.int32, (_EB, 1), 0)
    w = w * (rowid < e).astype(jnp.float32)
    wv_ref[...] = w[:, :_D]
    ws_ref[...] = w[:, _D:2 * _D]
    wr = w[:, 2 * _D:]
    wr0_ref[...] = wr * u[:, 0:1]
    wr1_ref[...] = wr * u[:, 1:2]
    wr2_ref[...] = wr * u[:, 2:3]


def _edgew(vecp, mw, mb, e):
    ep = vecp.shape[0]
    grid = ep // _EB
    out = jax.ShapeDtypeStruct((ep, _D), jnp.float32)
    return pl.pallas_call(
        functools.partial(_edgew_body, e),
        grid=(grid,),
        in_specs=[
            pl.BlockSpec((_EB, _D), lambda i: (i, 0)),
            pl.BlockSpec((_L, 3 * _D), lambda i: (0, 0)),
            pl.BlockSpec((3 * _D,), lambda i: (0,)),
        ],
        out_specs=[pl.BlockSpec((_EB, _D), lambda i: (i, 0))] * 5,
        out_shape=[out, out, out, out, out],
    )(vecp, mw, mb)


def _node2_body(nvt_ref, dv0_ref, dv1_ref, dv2_ref, dsp_ref, ns_ref,
                u1_ref, ub1_ref, u2_ref, ub2_ref,
                vn_ref, sn_ref, gv_ref, gs_ref, ga_ref):
    vn_ref[0] = nvt_ref[0] + dv0_ref[0] + dv0_ref[1]
    vn_ref[1] = nvt_ref[1] + dv1_ref[0] + dv1_ref[1]
    vn_ref[2] = nvt_ref[2] + dv2_ref[0] + dv2_ref[1]
    s_new = ns_ref[...] + dsp_ref[0] + dsp_ref[1]
    sn_ref[...] = s_new
    h = _ssp(jnp.dot(s_new, u1_ref[...],
                     preferred_element_type=jnp.float32) + ub1_ref[...])
    g = jnp.dot(h, u2_ref[...], preferred_element_type=jnp.float32) + ub2_ref[...]
    gv_ref[...] = g[:, :_D]
    gs_ref[...] = g[:, _D:2 * _D]
    ga_ref[...] = g[:, 2 * _D:]


def _node2(nvt, dv0p, dv1p, dv2p, dsp, ns, u1, ub1, u2, ub2):
    n = ns.shape[0]
    grid = n // _NB
    out = jax.ShapeDtypeStruct((n, _D), jnp.float32)
    p2 = pl.BlockSpec((2, _NB, _D), lambda i: (0, i, 0))
    return pl.pallas_call(
        _node2_body,
        grid=(grid,),
        in_specs=[
            pl.BlockSpec((3, _NB, _D), lambda i: (0, i, 0)),
            p2, p2, p2, p2,
            pl.BlockSpec((_NB, _D), lambda i: (i, 0)),
            pl.BlockSpec((_D, _D), lambda i: (0, 0)),
            pl.BlockSpec((_D,), lambda i: (0,)),
            pl.BlockSpec((_D, 3 * _D), lambda i: (0, 0)),
            pl.BlockSpec((3 * _D,), lambda i: (0,)),
        ],
        out_specs=[pl.BlockSpec((3, _NB, _D), lambda i: (0, i, 0)),
                   pl.BlockSpec((_NB, _D), lambda i: (i, 0)),
                   pl.BlockSpec((_NB, _D), lambda i: (i, 0)),
                   pl.BlockSpec((_NB, _D), lambda i: (i, 0)),
                   pl.BlockSpec((_NB, _D), lambda i: (i, 0))],
        out_shape=[jax.ShapeDtypeStruct((3, n, _D), jnp.float32),
                   out, out, out, out],
    )(nvt, dv0p, dv1p, dv2p, dsp, ns, u1, ub1, u2, ub2)


def _final_body(vn_ref, sn_ref, uv0_ref, uv1_ref, uv2_ref,
                smv_ref, sms_ref, sma_ref, deg_ref, vo_ref, so_ref):
    deg = jnp.maximum(deg_ref[0, :, 0:1] + deg_ref[1, :, 0:1], 1.0)
    inv = 1.0 / deg
    uv0 = (uv0_ref[0] + uv0_ref[1]) * inv
    uv1 = (uv1_ref[0] + uv1_ref[1]) * inv
    uv2 = (uv2_ref[0] + uv2_ref[1]) * inv
    avv = (smv_ref[0] + smv_ref[1]) * inv
    asv = (sms_ref[0] + sms_ref[1]) * inv
    ass = (sma_ref[0] + sma_ref[1]) * inv
    vo_ref[0] = vn_ref[0] + uv0 * avv
    vo_ref[1] = vn_ref[1] + uv1 * avv
    vo_ref[2] = vn_ref[2] + uv2 * avv
    s2 = uv0 * uv0 + uv1 * uv1 + uv2 * uv2
    so_ref[...] = sn_ref[...] + s2 / (s2 + _EPS) * asv + ass


def _final(vnt, sn, uv0p, uv1p, uv2p, smvp, smsp, smap, degp):
    n = sn.shape[0]
    grid = n // _NB
    p2 = pl.BlockSpec((2, _NB, _D), lambda i: (0, i, 0))
    return pl.pallas_call(
        _final_body,
        grid=(grid,),
        in_specs=[
            pl.BlockSpec((3, _NB, _D), lambda i: (0, i, 0)),
            pl.BlockSpec((_NB, _D), lambda i: (i, 0)),
            p2, p2, p2, p2, p2, p2, p2,
        ],
        out_specs=[pl.BlockSpec((3, _NB, _D), lambda i: (0, i, 0)),
                   pl.BlockSpec((_NB, _D), lambda i: (i, 0))],
        out_shape=[jax.ShapeDtypeStruct((3, n, _D), jnp.float32),
                   jax.ShapeDtypeStruct((n, _D), jnp.float32)],
    )(vnt, sn, uv0p, uv1p, uv2p, smvp, smsp, smap, degp)


_MESH = plsc.VectorSubcoreMesh(core_axis_name="c", subcore_axis_name="s")


def _wid_base():
    cid = lax.axis_index("c")
    sid = lax.axis_index("s")
    wid = sid * _NC + cid
    return cid, sid, wid * (_EP // _NW)


def _store_parts(cid, sid, n, acc, out_hbm):
    @pl.when(sid == 0)
    def _():
        @pl.when(cid == 0)
        def _():
            pltpu.sync_copy(acc.at[pl.ds(0, n)], out_hbm.at[0])

        @pl.when(cid == 1)
        def _():
            pltpu.sync_copy(acc.at[pl.ds(0, n)], out_hbm.at[1])


def _sc_vec(x_pad, src, dst):
    nwin = (_EP // _NW) // _K

    @functools.partial(
        pl.kernel, mesh=_MESH,
        out_type=jax.ShapeDtypeStruct((_EP, _D), jnp.float32),
        scratch_types=[
            pltpu.VMEM((_K,), jnp.int32),
            pltpu.VMEM((_K,), jnp.int32),
            pltpu.VMEM((_K, _D), jnp.float32),
            pltpu.VMEM((_K, _D), jnp.float32),
            pltpu.SemaphoreType.DMA,
        ],
    )
    def k(x_hbm, src_hbm, dst_hbm, out_hbm, sv, dv, xs, xd, sem):
        _, _, base0 = _wid_base()

        def body(i, _):
            base = base0 + i * _K
            pltpu.sync_copy(src_hbm.at[pl.ds(base, _K)], sv)
            pltpu.sync_copy(dst_hbm.at[pl.ds(base, _K)], dv)
            ca = pltpu.async_copy(x_hbm.at[sv], xs, sem)
            cb = pltpu.async_copy(x_hbm.at[dv], xd, sem)
            ca.wait()
            cb.wait()

            def row(r, _):
                sl = pl.ds(0, 16)
                xs[r, sl] = xs[r, sl] - xd[r, sl]
                return ()

            lax.fori_loop(0, _K, row, ())
            pltpu.sync_copy(xs, out_hbm.at[pl.ds(base, _K)])
            return ()

        lax.fori_loop(0, nwin, body, ())

    return k(x_pad, src, dst)


def _sc_pipe(kind, tabs, streams, src, dst, zeros_acc):
    n = zeros_acc.shape[0] - _SP
    nT, nS = len(tabs), len(streams)
    K = {"dv": 32, "ds": 64, "gacc": 128}[kind]
    nwin = (_EP // _NW) // K
    npair = nwin // 2
    wmax = nwin - 1

    scratch = []
    scratch += [pltpu.VMEM((K,), jnp.int32)] * 4
    scratch += [pltpu.VMEM((K, _D), jnp.float32)] * (2 * nT)
    scratch += [pltpu.VMEM((K, _D), jnp.float32)] * (2 * nS)
    scratch += [pltpu.VMEM_SHARED((10008, _D), jnp.float32)]
    scratch += [pltpu.SemaphoreType.DMA] * 4

    @functools.partial(pl.kernel, mesh=_MESH,
                       out_type=jax.ShapeDtypeStruct((_NC, n, _D), jnp.float32),
                       scratch_types=scratch)
    def k(*args):
        hbm = args[:nT + nS + 4]
        tab_h = hbm[:nT]
        st_h = hbm[nT:nT + nS]
        src_h, dst_h, z_h, out_h = hbm[nT + nS:]
        sc = args[nT + nS + 4:]
        sv = sc[0:2]
        dvv = sc[2:4]
        g = [sc[4 + 2 * t: 6 + 2 * t] for t in range(nT)]
        st = [sc[4 + 2 * nT + 2 * u: 6 + 2 * nT + 2 * u] for u in range(nS)]
        acc = sc[4 + 2 * nT + 2 * nS]
        semL = sc[5 + 2 * nT + 2 * nS: 7 + 2 * nT + 2 * nS]
        semG = sc[7 + 2 * nT + 2 * nS: 9 + 2 * nT + 2 * nS]

        cid, sid, base0 = _wid_base()
        base0 = (base0 // K) * K

        def issue_lin(p, w):
            base = base0 + w * K
            pltpu.async_copy(src_h.at[pl.ds(base, K)], sv[p], semL[p])
            pltpu.async_copy(dst_h.at[pl.ds(base, K)], dvv[p], semL[p])
            for u in range(nS):
                pltpu.async_copy(st_h[u].at[pl.ds(base, K)], st[u][p], semL[p])

        def drain_lin(p):
            pltpu.make_async_copy(src_h.at[pl.ds(0, K)], sv[p], semL[p]).wait()
            pltpu.make_async_copy(dst_h.at[pl.ds(0, K)], dvv[p], semL[p]).wait()
            for u in range(nS):
                pltpu.make_async_copy(st_h[u].at[pl.ds(0, K)], st[u][p],
                                      semL[p]).wait()

        def issue_gat(p):
            for t in range(nT):
                pltpu.async_copy(tab_h[t].at[sv[p]], g[t][p], semG[p])

        def drain_gat(p):
            for t in range(nT):
                pltpu.make_async_copy(tab_h[t].at[pl.ds(0, K)], g[t][p],
                                      semG[p]).wait()

        def compute(p):
            if kind == "gacc":
                return

            def row(r, _):
                for j in range(_D // 16):
                    sl = pl.ds(j * 16, 16)
                    if kind == "dv":
                        g[0][p][r, sl] = (g[0][p][r, sl] * st[0][p][r, sl]
                                          * g[2][p][r, sl])
                        g[1][p][r, sl] = g[1][p][r, sl] * st[1][p][r, sl]
                    else:
                        g[0][p][r, sl] = g[0][p][r, sl] * st[0][p][r, sl]
                return ()

            lax.fori_loop(0, K, row, ())

        def scatter(p):
            pltpu.sync_copy(g[0][p], acc.at[dvv[p]], add=True)
            if kind == "dv":
                pltpu.sync_copy(g[1][p], acc.at[dvv[p]], add=True)

        @pl.when(sid == 0)
        def _():
            pltpu.sync_copy(z_h, acc)

        plsc.subcore_barrier()

        issue_lin(0, 0)
        drain_lin(0)
        issue_gat(0)
        issue_lin(1, 1)

        def pair(i, _):
            for p in (0, 1):
                w = 2 * i + p
                q = 1 - p
                drain_gat(p)

                @pl.when(w < wmax)
                def _():
                    drain_lin(q)
                    issue_gat(q)

                compute(p)
                scatter(p)

                @pl.when(w + 2 <= wmax)
                def _():
                    issue_lin(p, w + 2)

            return ()

        lax.fori_loop(0, npair, pair, ())
        plsc.subcore_barrier()
        _store_parts(cid, sid, n, acc, out_h)

    return k(*tabs, *streams, src, dst, zeros_acc)


def _sc_dv(phi_v, phi_r, nvc, w_v, wrc, src, dst, zeros_acc):
    return _sc_pipe("dv", [phi_v, phi_r, nvc], [w_v, wrc], src, dst, zeros_acc)


def _sc_ds(phi_s, w_s, src, dst, zeros_acc):
    return _sc_pipe("ds", [phi_s], [w_s], src, dst, zeros_acc)


def _sc_gacc(tab, src, dst, zeros_acc):
    return _sc_pipe("gacc", [tab], [], src, dst, zeros_acc)


def _sc_deg(ones_k, dst, zeros_acc, n):
    nwin = (_EP // _NW) // _K

    @functools.partial(
        pl.kernel, mesh=_MESH,
        out_type=jax.ShapeDtypeStruct((_NC, n, _D), jnp.float32),
        scratch_types=[
            pltpu.VMEM((_K,), jnp.int32),
            pltpu.VMEM((_K, _D), jnp.float32),
            pltpu.VMEM_SHARED((10008, _D), jnp.float32),
        ],
    )
    def k(o_hbm, dst_hbm, z_hbm, out_hbm, dv, ones, acc):
        cid, sid, base0 = _wid_base()

        @pl.when(sid == 0)
        def _():
            pltpu.sync_copy(z_hbm, acc)

        pltpu.sync_copy(o_hbm, ones)
        plsc.subcore_barrier()

        def body(i, _):
            base = base0 + i * _K
            pltpu.sync_copy(dst_hbm.at[pl.ds(base, _K)], dv)
            pltpu.sync_copy(ones, acc.at[dv], add=True)
            return ()

        lax.fori_loop(0, nwin, body, ())
        plsc.subcore_barrier()
        _store_parts(cid, sid, n, acc, out_hbm)

    return k(ones_k, dst, zeros_acc)


def kernel(nv, ns, x, edge_index, ms1_w, ms1_b, ms2_w, ms2_b, mv_w, mv_b,
           us1_w, us1_b, us2_w, us2_b):
    n = nv.shape[0]
    e = edge_index.shape[1]
    src = edge_index[0]
    dst = edge_index[1]
    nvt = jnp.transpose(nv, (2, 0, 1))
    x_pad = jnp.pad(x, ((0, 0), (0, _D - 3)))
    zeros_acc = jnp.zeros((n + _SP, _D), jnp.float32)
    ones_k = jnp.ones((_K, _D), jnp.float32)

    pad = _EP - e
    fill_src = (jnp.arange(pad, dtype=jnp.int32) * 97) % n
    fill_dst = n + (jnp.arange(pad, dtype=jnp.int32) % _SP)
    srcp = jnp.concatenate([src, fill_src])
    dstv = jnp.concatenate([dst, fill_src])
    dstp = jnp.concatenate([dst, fill_dst])

    phi_v, phi_s, phi_r = _node1(ns, ms1_w, ms1_b, ms2_w, ms2_b)

    vecp = _sc_vec(x_pad, srcp, dstv)
    w_v, w_s, wr0, wr1, wr2 = _edgew(vecp, mv_w, mv_b, e)

    dv0p = _sc_dv(phi_v, phi_r, nvt[0], w_v, wr0, srcp, dstp, zeros_acc)
    dv1p = _sc_dv(phi_v, phi_r, nvt[1], w_v, wr1, srcp, dstp, zeros_acc)
    dv2p = _sc_dv(phi_v, phi_r, nvt[2], w_v, wr2, srcp, dstp, zeros_acc)
    dsp = _sc_ds(phi_s, w_s, srcp, dstp, zeros_acc)

    vnt, s_new, g_v, g_s, g_a = _node2(nvt, dv0p, dv1p, dv2p, dsp, ns,
                                       us1_w, us1_b, us2_w, us2_b)

    uv0p = _sc_gacc(vnt[0], srcp, dstp, zeros_acc)
    uv1p = _sc_gacc(vnt[1], srcp, dstp, zeros_acc)
    uv2p = _sc_gacc(vnt[2], srcp, dstp, zeros_acc)
    smvp = _sc_gacc(g_v, srcp, dstp, zeros_acc)
    smsp = _sc_gacc(g_s, srcp, dstp, zeros_acc)
    smap = _sc_gacc(g_a, srcp, dstp, zeros_acc)
    degp = _sc_deg(ones_k, dstp, zeros_acc, n)

    vot, s_out = _final(vnt, s_new, uv0p, uv1p, uv2p, smvp, smsp, smap, degp)
    return jnp.transpose(vot, (1, 2, 0)), s_out


# speedup vs baseline: 13.3245x; 1.0734x over previous
import functools
import jax
import jax.numpy as jnp
from jax import lax
from jax.experimental import pallas as pl
from jax.experimental.pallas import tpu as pltpu
from jax.experimental.pallas import tpu_sc as plsc

_NC = 2
_NS = 16
_NW = _NC * _NS
_EPS = 1e-05
_RC = 5.0
_L = 6
_LOG2 = 0.6931471805599453
_PI = 3.141592653589793
_D = 128
_NB = 1000
_EB = 2048
_EP = 163840
_K = 128
_KD = 64
_SP = 8


def _ssp(x):
    return jnp.maximum(x, 0.0) + jnp.log1p(jnp.exp(-jnp.abs(x))) - _LOG2


def _node1_body(ns_ref, w1_ref, b1_ref, w2_ref, b2_ref, pv_ref, ps_ref, pr_ref):
    h = _ssp(jnp.dot(ns_ref[...], w1_ref[...],
                     preferred_element_type=jnp.float32) + b1_ref[...])
    phi = jnp.dot(h, w2_ref[...], preferred_element_type=jnp.float32) + b2_ref[...]
    pv_ref[...] = phi[:, :_D]
    ps_ref[...] = phi[:, _D:2 * _D]
    pr_ref[...] = phi[:, 2 * _D:]


def _node1(ns, w1, b1, w2, b2):
    n = ns.shape[0]
    grid = n // _NB
    out = jax.ShapeDtypeStruct((n, _D), jnp.float32)
    return pl.pallas_call(
        _node1_body,
        grid=(grid,),
        in_specs=[
            pl.BlockSpec((_NB, _D), lambda i: (i, 0)),
            pl.BlockSpec((_D, _D), lambda i: (0, 0)),
            pl.BlockSpec((_D,), lambda i: (0,)),
            pl.BlockSpec((_D, 3 * _D), lambda i: (0, 0)),
            pl.BlockSpec((3 * _D,), lambda i: (0,)),
        ],
        out_specs=[pl.BlockSpec((_NB, _D), lambda i: (i, 0))] * 3,
        out_shape=[out, out, out],
    )(ns, w1, b1, w2, b2)


def _edgew_body(e, vec_ref, mw_ref, mb_ref, wv_ref, ws_ref,
                wr0_ref, wr1_ref, wr2_ref):
    i = pl.program_id(0)
    vec = vec_ref[...][:, :3]
    r2 = jnp.sum(vec * vec, axis=1, keepdims=True) + _EPS
    r = jnp.sqrt(r2)
    u = vec / r
    nn = lax.broadcasted_iota(jnp.int32, (1, _L), 1).astype(jnp.float32) + 1.0
    rbf = jnp.sqrt(2.0 / _RC) * jnp.sin(nn * (_PI / _RC) * r) / r
    z = jnp.dot(rbf, mw_ref[...], preferred_element_type=jnp.float32) + mb_ref[...]
    w = 0.5 * (jnp.cos((_PI / _RC) * z) + 1.0) * (z < _RC).astype(jnp.float32)
    rowid = i * _EB + lax.broadcasted_iota(jnp.int32, (_EB, 1), 0)
    w = w * (rowid < e).astype(jnp.float32)
    wv_ref[...] = w[:, :_D]
    ws_ref[...] = w[:, _D:2 * _D]
    wr = w[:, 2 * _D:]
    wr0_ref[...] = wr * u[:, 0:1]
    wr1_ref[...] = wr * u[:, 1:2]
    wr2_ref[...] = wr * u[:, 2:3]


def _edgew(vecp, mw, mb, e):
    ep = vecp.shape[0]
    grid = ep // _EB
    out = jax.ShapeDtypeStruct((ep, _D), jnp.float32)
    return pl.pallas_call(
        functools.partial(_edgew_body, e),
        grid=(grid,),
        in_specs=[
            pl.BlockSpec((_EB, _D), lambda i: (i, 0)),
            pl.BlockSpec((_L, 3 * _D), lambda i: (0, 0)),
            pl.BlockSpec((3 * _D,), lambda i: (0,)),
        ],
        out_specs=[pl.BlockSpec((_EB, _D), lambda i: (i, 0))] * 5,
        out_shape=[out, out, out, out, out],
    )(vecp, mw, mb)


def _node2_body(nvt_ref, dv0_ref, dv1_ref, dv2_ref, dsp_ref, ns_ref,
                u1_ref, ub1_ref, u2_ref, ub2_ref,
                vn_ref, sn_ref, gv_ref, gs_ref, ga_ref):
    vn_ref[0] = nvt_ref[0] + dv0_ref[0] + dv0_ref[1]
    vn_ref[1] = nvt_ref[1] + dv1_ref[0] + dv1_ref[1]
    vn_ref[2] = nvt_ref[2] + dv2_ref[0] + dv2_ref[1]
    s_new = ns_ref[...] + dsp_ref[0] + dsp_ref[1]
    sn_ref[...] = s_new
    h = _ssp(jnp.dot(s_new, u1_ref[...],
                     preferred_element_type=jnp.float32) + ub1_ref[...])
    g = jnp.dot(h, u2_ref[...], preferred_element_type=jnp.float32) + ub2_ref[...]
    gv_ref[...] = g[:, :_D]
    gs_ref[...] = g[:, _D:2 * _D]
    ga_ref[...] = g[:, 2 * _D:]


def _node2(nvt, dv0p, dv1p, dv2p, dsp, ns, u1, ub1, u2, ub2):
    n = ns.shape[0]
    grid = n // _NB
    out = jax.ShapeDtypeStruct((n, _D), jnp.float32)
    p2 = pl.BlockSpec((2, _NB, _D), lambda i: (0, i, 0))
    return pl.pallas_call(
        _node2_body,
        grid=(grid,),
        in_specs=[
            pl.BlockSpec((3, _NB, _D), lambda i: (0, i, 0)),
            p2, p2, p2, p2,
            pl.BlockSpec((_NB, _D), lambda i: (i, 0)),
            pl.BlockSpec((_D, _D), lambda i: (0, 0)),
            pl.BlockSpec((_D,), lambda i: (0,)),
            pl.BlockSpec((_D, 3 * _D), lambda i: (0, 0)),
            pl.BlockSpec((3 * _D,), lambda i: (0,)),
        ],
        out_specs=[pl.BlockSpec((3, _NB, _D), lambda i: (0, i, 0)),
                   pl.BlockSpec((_NB, _D), lambda i: (i, 0)),
                   pl.BlockSpec((_NB, _D), lambda i: (i, 0)),
                   pl.BlockSpec((_NB, _D), lambda i: (i, 0)),
                   pl.BlockSpec((_NB, _D), lambda i: (i, 0))],
        out_shape=[jax.ShapeDtypeStruct((3, n, _D), jnp.float32),
                   out, out, out, out],
    )(nvt, dv0p, dv1p, dv2p, dsp, ns, u1, ub1, u2, ub2)


def _final_body(vn_ref, sn_ref, uv0_ref, uv1_ref, uv2_ref,
                smv_ref, sms_ref, sma_ref, deg_ref, vo_ref, so_ref):
    deg = jnp.maximum(deg_ref[0, :, 0:1] + deg_ref[1, :, 0:1], 1.0)
    inv = 1.0 / deg
    uv0 = (uv0_ref[0] + uv0_ref[1]) * inv
    uv1 = (uv1_ref[0] + uv1_ref[1]) * inv
    uv2 = (uv2_ref[0] + uv2_ref[1]) * inv
    avv = (smv_ref[0] + smv_ref[1]) * inv
    asv = (sms_ref[0] + sms_ref[1]) * inv
    ass = (sma_ref[0] + sma_ref[1]) * inv
    vo_ref[0] = vn_ref[0] + uv0 * avv
    vo_ref[1] = vn_ref[1] + uv1 * avv
    vo_ref[2] = vn_ref[2] + uv2 * avv
    s2 = uv0 * uv0 + uv1 * uv1 + uv2 * uv2
    so_ref[...] = sn_ref[...] + s2 / (s2 + _EPS) * asv + ass


def _final(vnt, sn, uv0p, uv1p, uv2p, smvp, smsp, smap, degp):
    n = sn.shape[0]
    grid = n // _NB
    p2 = pl.BlockSpec((2, _NB, _D), lambda i: (0, i, 0))
    return pl.pallas_call(
        _final_body,
        grid=(grid,),
        in_specs=[
            pl.BlockSpec((3, _NB, _D), lambda i: (0, i, 0)),
            pl.BlockSpec((_NB, _D), lambda i: (i, 0)),
            p2, p2, p2, p2, p2, p2, p2,
        ],
        out_specs=[pl.BlockSpec((3, _NB, _D), lambda i: (0, i, 0)),
                   pl.BlockSpec((_NB, _D), lambda i: (i, 0))],
        out_shape=[jax.ShapeDtypeStruct((3, n, _D), jnp.float32),
                   jax.ShapeDtypeStruct((n, _D), jnp.float32)],
    )(vnt, sn, uv0p, uv1p, uv2p, smvp, smsp, smap, degp)


_MESH = plsc.VectorSubcoreMesh(core_axis_name="c", subcore_axis_name="s")


def _wid_base():
    cid = lax.axis_index("c")
    sid = lax.axis_index("s")
    wid = sid * _NC + cid
    return cid, sid, wid * (_EP // _NW)


def _store_parts(cid, sid, n, acc, out_hbm):
    @pl.when(sid == 0)
    def _():
        @pl.when(cid == 0)
        def _():
            pltpu.sync_copy(acc.at[pl.ds(0, n)], out_hbm.at[0])

        @pl.when(cid == 1)
        def _():
            pltpu.sync_copy(acc.at[pl.ds(0, n)], out_hbm.at[1])


def _sc_vec(x_pad, src, dst, dstp, ones_k, zeros_acc):
    n = zeros_acc.shape[0] - _SP
    nwin = (_EP // _NW) // _K

    @functools.partial(
        pl.kernel, mesh=_MESH,
        out_type=[jax.ShapeDtypeStruct((_EP, _D), jnp.float32),
                  jax.ShapeDtypeStruct((_NC, n, _D), jnp.float32)],
        scratch_types=[
            pltpu.VMEM((_K,), jnp.int32),
            pltpu.VMEM((_K,), jnp.int32),
            pltpu.VMEM((_K,), jnp.int32),
            pltpu.VMEM((_K, _D), jnp.float32),
            pltpu.VMEM((_K, _D), jnp.float32),
            pltpu.VMEM((_K, _D), jnp.float32),
            pltpu.VMEM_SHARED((10008, _D), jnp.float32),
            pltpu.SemaphoreType.DMA,
        ],
    )
    def k(x_hbm, src_hbm, dst_hbm, dstp_hbm, o_hbm, z_hbm, out_hbm, deg_hbm,
          sv, dv, dp, xs, xd, ones, acc, sem):
        cid, sid, base0 = _wid_base()

        @pl.when(sid == 0)
        def _():
            pltpu.sync_copy(z_hbm, acc)

        pltpu.sync_copy(o_hbm, ones)
        plsc.subcore_barrier()

        def body(i, _):
            base = base0 + i * _K
            pltpu.sync_copy(src_hbm.at[pl.ds(base, _K)], sv)
            pltpu.sync_copy(dst_hbm.at[pl.ds(base, _K)], dv)
            pltpu.sync_copy(dstp_hbm.at[pl.ds(base, _K)], dp)
            ca = pltpu.async_copy(x_hbm.at[sv], xs, sem)
            cb = pltpu.async_copy(x_hbm.at[dv], xd, sem)
            pltpu.sync_copy(ones, acc.at[dp], add=True)
            ca.wait()
            cb.wait()

            def row(r, _):
                sl = pl.ds(0, 16)
                xs[r, sl] = xs[r, sl] - xd[r, sl]
                return ()

            lax.fori_loop(0, _K, row, ())
            pltpu.sync_copy(xs, out_hbm.at[pl.ds(base, _K)])
            return ()

        lax.fori_loop(0, nwin, body, ())
        plsc.subcore_barrier()
        _store_parts(cid, sid, n, acc, deg_hbm)

    return k(x_pad, src, dst, dstp, ones_k, zeros_acc)


def _sc_pipe(kind, tabs, streams, src, dst, zeros_acc):
    n = zeros_acc.shape[0] - _SP
    nT, nS = len(tabs), len(streams)
    K = {"dv": 32, "ds": 64, "gacc": 128}[kind]
    nwin = (_EP // _NW) // K
    npair = nwin // 2
    wmax = nwin - 1

    scratch = []
    scratch += [pltpu.VMEM((K,), jnp.int32)] * 6
    scratch += [pltpu.VMEM((K, _D), jnp.float32)] * (2 * nT)
    scratch += [pltpu.VMEM((K, _D), jnp.float32)] * (2 * nS)
    scratch += [pltpu.VMEM_SHARED((10008, _D), jnp.float32)]
    scratch += [pltpu.SemaphoreType.DMA] * 6

    @functools.partial(pl.kernel, mesh=_MESH,
                       out_type=jax.ShapeDtypeStruct((_NC, n, _D), jnp.float32),
                       scratch_types=scratch)
    def k(*args):
        hbm = args[:nT + nS + 4]
        tab_h = hbm[:nT]
        st_h = hbm[nT:nT + nS]
        src_h, dst_h, z_h, out_h = hbm[nT + nS:]
        sc = args[nT + nS + 4:]
        sv = sc[0:2]
        dvv = sc[2:4]
        sdv = sc[4:6]
        g = [sc[6 + 2 * t: 8 + 2 * t] for t in range(nT)]
        st = [sc[6 + 2 * nT + 2 * u: 8 + 2 * nT + 2 * u] for u in range(nS)]
        acc = sc[6 + 2 * nT + 2 * nS]
        semL = sc[7 + 2 * nT + 2 * nS: 9 + 2 * nT + 2 * nS]
        semG = sc[9 + 2 * nT + 2 * nS: 11 + 2 * nT + 2 * nS]
        semS = sc[11 + 2 * nT + 2 * nS: 13 + 2 * nT + 2 * nS]

        cid, sid, base0 = _wid_base()
        base0 = (base0 // K) * K

        def issue_lin(p, w):
            base = base0 + w * K
            pltpu.async_copy(src_h.at[pl.ds(base, K)], sv[p], semL[p])
            pltpu.async_copy(dst_h.at[pl.ds(base, K)], dvv[p], semL[p])
            for u in range(nS):
                pltpu.async_copy(st_h[u].at[pl.ds(base, K)], st[u][p], semL[p])

        def drain_lin(p):
            pltpu.make_async_copy(src_h.at[pl.ds(0, K)], sv[p], semL[p]).wait()
            pltpu.make_async_copy(dst_h.at[pl.ds(0, K)], dvv[p], semL[p]).wait()
            for u in range(nS):
                pltpu.make_async_copy(st_h[u].at[pl.ds(0, K)], st[u][p],
                                      semL[p]).wait()

        def issue_gat(p):
            for t in range(nT):
                pltpu.async_copy(tab_h[t].at[sv[p]], g[t][p], semG[p])

        def drain_gat(p):
            for t in range(nT):
                pltpu.make_async_copy(tab_h[t].at[pl.ds(0, K)], g[t][p],
                                      semG[p]).wait()

        def compute(p):
            if kind == "gacc":
                return

            def row(r, _):
                for j in range(_D // 16):
                    sl = pl.ds(j * 16, 16)
                    if kind == "dv":
                        g[0][p][r, sl] = (g[0][p][r, sl] * st[0][p][r, sl]
                                          * g[2][p][r, sl])
                        g[1][p][r, sl] = g[1][p][r, sl] * st[1][p][r, sl]
                    else:
                        g[0][p][r, sl] = g[0][p][r, sl] * st[0][p][r, sl]
                return ()

            lax.fori_loop(0, K, row, ())

        def copy_idx(p):
            for j in range(K // 16):
                sl = pl.ds(j * 16, 16)
                sdv[p][sl] = dvv[p][sl]

        def scatter(p):
            pltpu.async_copy(g[0][p], acc.at[sdv[p]], semS[p], add=True)
            if kind == "dv":
                pltpu.async_copy(g[1][p], acc.at[sdv[p]], semS[p], add=True)

        def drain_scat(p):
            pltpu.make_async_copy(tab_h[0].at[pl.ds(0, K)], g[0][p],
                                  semS[p]).wait()
            if kind == "dv":
                pltpu.make_async_copy(tab_h[0].at[pl.ds(0, K)], g[1][p],
                                      semS[p]).wait()

        @pl.when(sid == 0)
        def _():
            pltpu.sync_copy(z_h, acc)

        plsc.subcore_barrier()

        issue_lin(0, 0)
        drain_lin(0)
        copy_idx(0)
        issue_gat(0)
        issue_lin(1, 1)

        def pair(i, _):
            for p in (0, 1):
                w = 2 * i + p
                q = 1 - p
                drain_gat(p)

                @pl.when(w < wmax)
                def _():
                    drain_lin(q)

                    @pl.when(w >= 1)
                    def _():
                        drain_scat(q)

                    copy_idx(q)
                    issue_gat(q)

                compute(p)
                scatter(p)

                @pl.when(w + 2 <= wmax)
                def _():
                    issue_lin(p, w + 2)

            return ()

        lax.fori_loop(0, npair, pair, ())
        drain_scat(0)
        drain_scat(1)
        plsc.subcore_barrier()
        _store_parts(cid, sid, n, acc, out_h)

    return k(*tabs, *streams, src, dst, zeros_acc)


def _sc_dv(phi_v, phi_r, nvc, w_v, wrc, src, dst, zeros_acc):
    return _sc_pipe("dv", [phi_v, phi_r, nvc], [w_v, wrc], src, dst, zeros_acc)


def _sc_ds(phi_s, w_s, src, dst, zeros_acc):
    return _sc_pipe("ds", [phi_s], [w_s], src, dst, zeros_acc)


def _sc_gacc(tab, src, dst, zeros_acc):
    return _sc_pipe("gacc", [tab], [], src, dst, zeros_acc)


def kernel(nv, ns, x, edge_index, ms1_w, ms1_b, ms2_w, ms2_b, mv_w, mv_b,
           us1_w, us1_b, us2_w, us2_b):
    n = nv.shape[0]
    e = edge_index.shape[1]
    src = edge_index[0]
    dst = edge_index[1]
    nvt = jnp.transpose(nv, (2, 0, 1))
    x_pad = jnp.pad(x, ((0, 0), (0, _D - 3)))
    zeros_acc = jnp.zeros((n + _SP, _D), jnp.float32)
    ones_k = jnp.ones((_K, _D), jnp.float32)

    pad = _EP - e
    fill_src = (jnp.arange(pad, dtype=jnp.int32) * 97) % n
    fill_dst = n + (jnp.arange(pad, dtype=jnp.int32) % _SP)
    srcp = jnp.concatenate([src, fill_src])
    dstv = jnp.concatenate([dst, fill_src])
    dstp = jnp.concatenate([dst, fill_dst])

    phi_v, phi_s, phi_r = _node1(ns, ms1_w, ms1_b, ms2_w, ms2_b)

    vecp, degp = _sc_vec(x_pad, srcp, dstv, dstp, ones_k, zeros_acc)
    w_v, w_s, wr0, wr1, wr2 = _edgew(vecp, mv_w, mv_b, e)

    dv0p = _sc_dv(phi_v, phi_r, nvt[0], w_v, wr0, srcp, dstp, zeros_acc)
    dv1p = _sc_dv(phi_v, phi_r, nvt[1], w_v, wr1, srcp, dstp, zeros_acc)
    dv2p = _sc_dv(phi_v, phi_r, nvt[2], w_v, wr2, srcp, dstp, zeros_acc)
    dsp = _sc_ds(phi_s, w_s, srcp, dstp, zeros_acc)

    vnt, s_new, g_v, g_s, g_a = _node2(nvt, dv0p, dv1p, dv2p, dsp, ns,
                                       us1_w, us1_b, us2_w, us2_b)

    uv0p = _sc_gacc(vnt[0], srcp, dstp, zeros_acc)
    uv1p = _sc_gacc(vnt[1], srcp, dstp, zeros_acc)
    uv2p = _sc_gacc(vnt[2], srcp, dstp, zeros_acc)
    smvp = _sc_gacc(g_v, srcp, dstp, zeros_acc)
    smsp = _sc_gacc(g_s, srcp, dstp, zeros_acc)
    smap = _sc_gacc(g_a, srcp, dstp, zeros_acc)

    vot, s_out = _final(vnt, s_new, uv0p, uv1p, uv2p, smvp, smsp, smap, degp)
    return jnp.transpose(vot, (1, 2, 0)), s_out
